# Initial kernel scaffold; baseline (speedup 1.0000x reference)
import jax
import jax.numpy as jnp
from jax.experimental import pallas as pl


def kernel(x, edge_index, edge_attr, batch, W0, b0, g0, t0, W1, b1, g1, t1, W2, b2, g2, t2, mW1, mb1, mW2, mb2):
    raise NotImplementedError("write your pallas kernel here")


# speedup vs baseline: 1.5614x; 1.5614x over previous
import functools

import jax
import jax.numpy as jnp
from jax import lax
from jax.experimental import pallas as pl
from jax.experimental.pallas import tpu as pltpu
from jax.experimental.pallas import tpu_sc as plsc

N = 10000
E = 320000
H = 256
HH = 128
NSUB = 16
NCORE = 2
CHUNK = 128
EP = E + CHUNK
NPT = 624
NPT_LAST = 640
ACC_ROWS = 648
DUMP_ROW = 647

_MESH = plsc.VectorSubcoreMesh(core_axis_name="c", subcore_axis_name="s")


@functools.partial(
    pl.kernel,
    out_type=jax.ShapeDtypeStruct((NCORE * N, HH), jnp.float32),
    mesh=_MESH,
    scratch_types=[
        pltpu.VMEM((CHUNK,), jnp.int32),
        pltpu.VMEM((CHUNK,), jnp.int32),
        pltpu.VMEM((CHUNK,), jnp.float32),
        pltpu.VMEM((16,), jnp.int32),
        pltpu.VMEM((16,), jnp.float32),
        pltpu.VMEM((CHUNK, HH), jnp.float32),
        pltpu.VMEM((ACC_ROWS, HH), jnp.float32),
        pltpu.SemaphoreType.DMA,
    ],
)
def _sc_message(hcat_hbm, srcg_hbm, dst_hbm, norm_hbm, dinv_hbm,
                rowptr_hbm, accs_hbm,
                sidx, didx, wv, rpb, dvb, buf, acc, sem):
    c = lax.axis_index("c")
    s = lax.axis_index("s")
    nstart = s * NPT
    nsize = jnp.where(s == NSUB - 1, NPT_LAST, NPT)

    def zrow(r, carry):
        for j in range(HH // 16):
            acc[r, pl.ds(j * 16, 16)] = jnp.zeros((16,), jnp.float32)
        return carry
    lax.fori_loop(0, ACC_ROWS, zrow, None)

    pltpu.sync_copy(rowptr_hbm.at[pl.ds(nstart, 16)], rpb)
    estart = rpb[pl.ds(0, 16)][0]
    eoff = jnp.where(s == NSUB - 1, N, nstart + NPT)
    pltpu.sync_copy(rowptr_hbm.at[pl.ds(eoff, 16)], rpb)
    eend = rpb[pl.ds(0, 16)][0]

    abase = pl.multiple_of((estart // 8) * 8, 8)
    nch = (eend - abase + (CHUNK - 1)) // CHUNK

    def chunk_body(k, carry):
        off = abase + k * CHUNK
        pltpu.sync_copy(srcg_hbm.at[pl.ds(c * EP + off, CHUNK)], sidx)
        pltpu.sync_copy(dst_hbm.at[pl.ds(off, CHUNK)], didx)
        pltpu.sync_copy(norm_hbm.at[pl.ds(off, CHUNK)], wv)

        pltpu.async_copy(hcat_hbm.at[sidx], buf, sem).wait()

        def abody(g, carry2):
            sl = pl.ds(g * 16, 16)
            ngrp = wv[sl]
            dgrp = didx[sl]
            for kk in range(16):
                e = g * 16 + kk
                ge = off + e
                valid = jnp.logical_and(ge >= estart, ge < eend)
                tr = jnp.where(valid, dgrp[kk] - nstart, DUMP_ROW)
                ns = ngrp[kk]
                for j in range(HH // 16):
                    fs = pl.ds(j * 16, 16)
                    acc[tr, fs] = acc[tr, fs] + buf[e, fs] * ns
            return carry2
        lax.fori_loop(0, CHUNK // 16, abody, None)
        return carry

    lax.fori_loop(0, nch, chunk_body, None)

    def sbody(g, carry):
        rbase = g * 16
        pltpu.sync_copy(hcat_hbm.at[pl.ds(c * N + nstart + rbase, 16)],
                        buf.at[pl.ds(0, 16)])
        pltpu.sync_copy(dinv_hbm.at[pl.ds(nstart + rbase, 16)], dvb)
        dv = dvb[pl.ds(0, 16)]
        nsv = (dv * 1.0) * dv
        for kk in range(16):
            row = rbase + kk

            @pl.when(row < nsize)
            def _():
                ns = nsv[kk]
                for j in range(HH // 16):
                    fs = pl.ds(j * 16, 16)
                    acc[row, fs] = acc[row, fs] + buf[kk, fs] * ns
        return carry
    lax.fori_loop(0, NPT_LAST // 16, sbody, None)

    @pl.when(s < NSUB - 1)
    def _wb():
        pltpu.sync_copy(acc.at[pl.ds(0, NPT)],
                        accs_hbm.at[pl.ds(c * N + nstart, NPT)])

    @pl.when(s == NSUB - 1)
    def _wb_last():
        pltpu.sync_copy(acc.at[pl.ds(0, NPT_LAST)],
                        accs_hbm.at[pl.ds(c * N + nstart, NPT_LAST)])


BN_BLK = 1000
NB = N // BN_BLK


def _mm_body(z_ref, w_ref, o_ref):
    o_ref[...] = jnp.dot(z_ref[...], w_ref[...],
                         preferred_element_type=jnp.float32)[None]


def _tc_mm(z, W):
    d_in = z.shape[1]
    return pl.pallas_call(
        _mm_body,
        grid=(NB, 2),
        in_specs=[
            pl.BlockSpec((BN_BLK, d_in), lambda i, ci: (i, 0)),
            pl.BlockSpec((d_in, HH), lambda i, ci: (0, ci)),
        ],
        out_specs=pl.BlockSpec((1, BN_BLK, HH), lambda i, ci: (ci, i, 0)),
        out_shape=jax.ShapeDtypeStruct((2, N, HH), jnp.float32),
    )(z, W)


def _act_body(acc_ref, b_ref, v_ref):
    a = acc_ref[...]
    u = jnp.concatenate([a[0], a[1]], axis=1) + b_ref[...]
    v_ref[...] = jnp.where(u >= 0, u, jnp.float32(0.2) * u)


def _tc_act(accs, b):
    return pl.pallas_call(
        _act_body,
        grid=(NB,),
        in_specs=[
            pl.BlockSpec((2, BN_BLK, HH), lambda i: (0, i, 0)),
            pl.BlockSpec((1, H), lambda i: (0, 0)),
        ],
        out_specs=pl.BlockSpec((BN_BLK, H), lambda i: (i, 0)),
        out_shape=jax.ShapeDtypeStruct((N, H), jnp.float32),
    )(accs, b.reshape(1, H))


def _bnmm_body(v_ref, mu_ref, var_ref, g_ref, t_ref, w_ref, o_ref):
    z = g_ref[...] * (v_ref[...] - mu_ref[...]) / jnp.sqrt(
        var_ref[...] + 1e-5) + t_ref[...]
    o_ref[...] = jnp.dot(z, w_ref[...],
                         preferred_element_type=jnp.float32)[None]


def _tc_bnmm(v, mu, var, g, t, W):
    return pl.pallas_call(
        _bnmm_body,
        grid=(NB, 2),
        in_specs=[
            pl.BlockSpec((BN_BLK, H), lambda i, ci: (i, 0)),
            pl.BlockSpec((1, H), lambda i, ci: (0, 0)),
            pl.BlockSpec((1, H), lambda i, ci: (0, 0)),
            pl.BlockSpec((1, H), lambda i, ci: (0, 0)),
            pl.BlockSpec((1, H), lambda i, ci: (0, 0)),
            pl.BlockSpec((H, HH), lambda i, ci: (0, ci)),
        ],
        out_specs=pl.BlockSpec((1, BN_BLK, HH), lambda i, ci: (ci, i, 0)),
        out_shape=jax.ShapeDtypeStruct((2, N, HH), jnp.float32),
    )(v, mu.reshape(1, H), var.reshape(1, H), g.reshape(1, H),
      t.reshape(1, H), W)


def kernel(x, edge_index, edge_attr, batch, W0, b0, g0, t0, W1, b1, g1, t1,
           W2, b2, g2, t2, mW1, mb1, mW2, mb2):
    src = edge_index[0]
    dst = edge_index[1]
    ew = jnp.abs(edge_attr)

    loop = jnp.arange(N, dtype=edge_index.dtype)
    col = jnp.concatenate([dst, loop])
    wfull = jnp.concatenate([ew, jnp.ones((N,), x.dtype)])
    deg = jnp.zeros((N,), x.dtype).at[col].add(wfull)
    dinv = 1.0 / jnp.sqrt(deg)

    esort = jnp.argsort(dst, stable=True)
    ssrc = src[esort]
    sdst = dst[esort]
    sw = ew[esort]
    snorm = dinv[ssrc] * sw * dinv[sdst]
    rowptr = jnp.searchsorted(sdst, jnp.arange(N + 1, dtype=jnp.int32),
                              side="left").astype(jnp.int32)
    rowptr_p = jnp.concatenate(
        [rowptr, jnp.full((15,), E, jnp.int32)])
    padi = jnp.zeros((CHUNK,), jnp.int32)
    sdst_p = jnp.concatenate([sdst, padi])
    snorm_p = jnp.concatenate([snorm, jnp.zeros((CHUNK,), jnp.float32)])
    srcg = jnp.concatenate([ssrc, padi, ssrc + N, padi])

    hsp = _tc_mm(x, W0)
    params = ((b0, g0, t0), (b1, g1, t1), (b2, g2, t2))
    Ws = (W1, W2)
    v = None
    for l in range(3):
        b, g, t = params[l]
        accs = _sc_message(hsp.reshape(2 * N, HH), srcg, sdst_p,
                           snorm_p, dinv, rowptr_p)
        v = _tc_act(accs.reshape(2, N, HH), b)
        mu = jnp.mean(v, axis=0)
        var = jnp.var(v, axis=0)
        if l < 2:
            hsp = _tc_bnmm(v, mu, var, g, t, Ws[l])

    z3 = g2 * (v - mu) / jnp.sqrt(var + 1e-5) + t2
    sseg = jax.ops.segment_sum(z3, batch, num_segments=1)
    cnt = jax.ops.segment_sum(jnp.ones((N,), z3.dtype), batch, num_segments=1)
    pooled = sseg / cnt[:, None]
    hmid = jax.nn.relu(pooled @ mW1 + mb1)
    return hmid @ mW2 + mb2
